# baseline (speedup 1.0000x reference)
import functools

import jax
import jax.numpy as jnp
from jax import lax
from jax.experimental import pallas as pl
from jax.experimental.pallas import tpu as pltpu
from jax.experimental.pallas import tpu_sc as plsc

N = 16384
D = 64
T_PAD = 64
IDX_CHUNK = 128


def _make_kernel():
    info = plsc.get_sparse_core_info()
    NC, NS, L = info.num_cores, info.num_subcores, info.num_lanes
    NW = NC * NS
    BPW = N // NW
    NCHUNK = BPW // IDX_CHUNK

    mesh = plsc.VectorSubcoreMesh(core_axis_name="c", subcore_axis_name="s")

    @functools.partial(
        pl.kernel,
        mesh=mesh,
        out_type=jax.ShapeDtypeStruct((N, D), jnp.float32),
        compiler_params=pltpu.CompilerParams(
            needs_layout_passes=False, use_tc_tiling_on_sc=False),
        scratch_types=[
            pltpu.VMEM((NCHUNK, IDX_CHUNK), jnp.int32),
            pltpu.VMEM((BPW, D), jnp.float32),
            pltpu.VMEM((T_PAD * T_PAD,), jnp.float32),
            pltpu.VMEM((T_PAD,), jnp.float32),
            pltpu.VMEM((BPW,), jnp.int32),
            pltpu.VMEM((BPW,), jnp.float32),
            pltpu.VMEM((BPW,), jnp.float32),
            pltpu.SemaphoreType.DMA,
        ],
    )
    def emb_kernel(x_hbm, maskf_hbm, tid_hbm, emb_hbm, tag_hbm, out_hbm,
                   idx_v, rows_v, tag_v, sums_v, tid_v, maskf_v, scale_v, sem):
        wid = lax.axis_index("s") * NC + lax.axis_index("c")
        base = wid * BPW

        for g in range(NCHUNK):
            pltpu.sync_copy(x_hbm.at[pl.ds(base + g * IDX_CHUNK, IDX_CHUNK)],
                            idx_v.at[g])
        gathers = [
            pltpu.async_copy(emb_hbm.at[idx_v.at[g]],
                             rows_v.at[pl.ds(g * IDX_CHUNK, IDX_CHUNK)],
                             sem)
            for g in range(NCHUNK)
        ]

        pltpu.sync_copy(tag_hbm, tag_v)
        pltpu.sync_copy(tid_hbm.at[pl.ds(base, BPW)], tid_v)
        pltpu.sync_copy(maskf_hbm.at[pl.ds(base, BPW)], maskf_v)

        lanes = lax.iota(jnp.int32, L)
        for g in range(T_PAD // L):
            t_vec = lanes + (g * L)
            row_base = t_vec * T_PAD
            acc = jnp.zeros((L,), jnp.float32)
            for c in range(T_PAD):
                acc = acc + plsc.load_gather(tag_v, [row_base + c])
            plsc.store_scatter(sums_v, [t_vec], acc)

        for g in range(BPW // L):
            sl = pl.ds(g * L, L)
            tid = tid_v[sl]
            s = plsc.load_gather(sums_v, [tid]) * maskf_v[sl]
            scale_v[sl] = s

        for c in gathers:
            c.wait()

        def scale_rows(i, _):
            for r in range(4):
                n = i * 4 + r
                sbc = plsc.load_gather(scale_v, [jnp.full((L,), n, jnp.int32)])
                for j in range(D // L):
                    sl = pl.ds(j * L, L)
                    rows_v[n, sl] = rows_v[n, sl] * sbc
            return _

        lax.fori_loop(0, BPW // 4, scale_rows, None)

        pltpu.sync_copy(rows_v, out_hbm.at[pl.ds(base, BPW)])

    return emb_kernel


_emb_kernel = _make_kernel()


@jax.jit
def kernel(x, mask, tag_id, emb_table, tag_table):
    x = x.astype(jnp.int32)
    maskf = mask.astype(jnp.float32)
    tag_id = tag_id.astype(jnp.int32)
    t, td = tag_table.shape
    tag_pad = jnp.zeros((T_PAD, T_PAD), jnp.float32).at[:t, :td].set(tag_table)
    return _emb_kernel(x, maskf, tag_id, emb_table, tag_pad.reshape(-1))


# speedup vs baseline: 1.6953x; 1.6953x over previous
import functools

import jax
import jax.numpy as jnp
from jax import lax
from jax.experimental import pallas as pl
from jax.experimental.pallas import tpu as pltpu
from jax.experimental.pallas import tpu_sc as plsc

N = 16384
D = 64
T_PAD = 64


def _make_kernel():
    info = plsc.get_sparse_core_info()
    NC, NS, L = info.num_cores, info.num_subcores, info.num_lanes
    NW = NC * NS
    BPW = N // NW

    mesh = plsc.VectorSubcoreMesh(core_axis_name="c", subcore_axis_name="s")

    @functools.partial(
        pl.kernel,
        mesh=mesh,
        out_type=jax.ShapeDtypeStruct((N, D), jnp.float32),
        compiler_params=pltpu.CompilerParams(needs_layout_passes=False),
        scratch_types=[
            pltpu.VMEM((BPW,), jnp.int32),
            pltpu.VMEM((BPW, D), jnp.float32),
            pltpu.VMEM((T_PAD * T_PAD,), jnp.float32),
            pltpu.VMEM((T_PAD,), jnp.float32),
            pltpu.VMEM((BPW,), jnp.int32),
            pltpu.VMEM((BPW,), jnp.float32),
            pltpu.VMEM((BPW,), jnp.float32),
            pltpu.SemaphoreType.DMA,
        ],
    )
    def emb_kernel(x_hbm, maskf_hbm, tid_hbm, tag_hbm, emb_hbm, out_hbm,
                   idx_v, rows_v, tag_v, sums_v, tid_v, maskf_v,
                   scale_v, sem):
        wid = lax.axis_index("s") * NC + lax.axis_index("c")
        base = wid * BPW

        pltpu.sync_copy(x_hbm.at[pl.ds(base, BPW)], idx_v)

        def fire(g, _):
            iv = idx_v[pl.ds(g * L, L)]
            for r in range(L):
                pltpu.async_copy(emb_hbm.at[pl.ds(iv[r], 1)],
                                 rows_v.at[pl.ds(g * L + r, 1)], sem)
            return _

        lax.fori_loop(0, BPW // L, fire, None)

        pltpu.sync_copy(tag_hbm, tag_v)
        pltpu.sync_copy(tid_hbm.at[pl.ds(base, BPW)], tid_v)
        pltpu.sync_copy(maskf_hbm.at[pl.ds(base, BPW)], maskf_v)

        lanes = lax.iota(jnp.int32, L)
        for g in range(T_PAD // L):
            t_vec = lanes + (g * L)
            row_base = t_vec * T_PAD
            acc = jnp.zeros((L,), jnp.float32)
            for c in range(T_PAD):
                acc = acc + plsc.load_gather(tag_v, [row_base + c])
            plsc.store_scatter(sums_v, [t_vec], acc)

        for g in range(BPW // L):
            sl = pl.ds(g * L, L)
            s = plsc.load_gather(sums_v, [tid_v[sl]]) * maskf_v[sl]
            scale_v[sl] = s

        def drain(r, _):
            pltpu.make_async_copy(emb_hbm.at[pl.ds(0, 1)],
                                  rows_v.at[pl.ds(r, 1)], sem).wait()
            return _

        lax.fori_loop(0, BPW, drain, None)

        def scale_rows(i, _):
            for r in range(4):
                n = i * 4 + r
                sbc = plsc.load_gather(scale_v, [jnp.full((L,), n, jnp.int32)])
                for j in range(D // L):
                    sl = pl.ds(j * L, L)
                    rows_v[n, sl] = rows_v[n, sl] * sbc
            return _

        lax.fori_loop(0, BPW // 4, scale_rows, None)

        pltpu.sync_copy(rows_v, out_hbm.at[pl.ds(base, BPW)])

    return emb_kernel


_emb_kernel = _make_kernel()


@jax.jit
def kernel(x, mask, tag_id, emb_table, tag_table):
    x = x.astype(jnp.int32)
    maskf = mask.astype(jnp.float32)
    tag_id = tag_id.astype(jnp.int32)
    t, td = tag_table.shape
    tag_pad = jnp.zeros((T_PAD, T_PAD), jnp.float32).at[:t, :td].set(tag_table)
    return _emb_kernel(x, maskf, tag_id, tag_pad.reshape(-1), emb_table)


# speedup vs baseline: 2.2111x; 1.3043x over previous
import functools

import jax
import jax.numpy as jnp
from jax import lax
from jax.experimental import pallas as pl
from jax.experimental.pallas import tpu as pltpu
from jax.experimental.pallas import tpu_sc as plsc

N = 16384
V = 1000000
D = 64
SLAB = 8
T_PAD = 64
CH = 64


def _make_kernel():
    info = plsc.get_sparse_core_info()
    NC, NS, L = info.num_cores, info.num_subcores, info.num_lanes
    NW = NC * NS
    BPW = N // NW
    NCH = BPW // CH

    mesh = plsc.VectorSubcoreMesh(core_axis_name="c", subcore_axis_name="s")

    @functools.partial(
        pl.kernel,
        mesh=mesh,
        out_type=jax.ShapeDtypeStruct((N, D), jnp.float32),
        compiler_params=pltpu.CompilerParams(needs_layout_passes=False),
        scratch_types=[
            pltpu.VMEM((BPW,), jnp.int32),
            pltpu.VMEM((BPW,), jnp.int32),
            pltpu.VMEM((CH, SLAB, D), jnp.float32),
            pltpu.VMEM((CH, D), jnp.float32),
            pltpu.VMEM((T_PAD * T_PAD,), jnp.float32),
            pltpu.VMEM((T_PAD,), jnp.float32),
            pltpu.VMEM((BPW,), jnp.int32),
            pltpu.VMEM((BPW,), jnp.float32),
            pltpu.VMEM((BPW,), jnp.float32),
            pltpu.SemaphoreType.DMA,
        ],
    )
    def emb_kernel(x_hbm, maskf_hbm, tid_hbm, tag_hbm, emb_hbm, out_hbm,
                   idx_v, sidx_v, slab_v, stage_v, tag_v, sums_v, tid_v,
                   maskf_v, scale_v, gsem):
        wid = lax.axis_index("s") * NC + lax.axis_index("c")
        base = wid * BPW

        pltpu.sync_copy(x_hbm.at[pl.ds(base, BPW)], idx_v)
        for g in range(BPW // L):
            sl = pl.ds(g * L, L)
            sidx_v[sl] = lax.shift_right_logical(idx_v[sl], 3)

        def fire(c):
            def fire16(g, _):
                iv = sidx_v[pl.ds(c * CH + g * L, L)]
                for r in range(L):
                    pltpu.async_copy(emb_hbm.at[pl.ds(iv[r], 1)],
                                     slab_v.at[pl.ds(g * L + r, 1)], gsem)
                return _
            lax.fori_loop(0, CH // L, fire16, None)

        fire(0)

        pltpu.sync_copy(tag_hbm, tag_v)
        pltpu.sync_copy(tid_hbm.at[pl.ds(base, BPW)], tid_v)
        pltpu.sync_copy(maskf_hbm.at[pl.ds(base, BPW)], maskf_v)

        lanes = lax.iota(jnp.int32, L)
        for g in range(T_PAD // L):
            t_vec = lanes + (g * L)
            row_base = t_vec * T_PAD
            acc = jnp.zeros((L,), jnp.float32)
            for c in range(T_PAD):
                acc = acc + plsc.load_gather(tag_v, [row_base + c])
            plsc.store_scatter(sums_v, [t_vec], acc)

        for g in range(BPW // L):
            sl = pl.ds(g * L, L)
            scale_v[sl] = plsc.load_gather(sums_v, [tid_v[sl]]) * maskf_v[sl]

        for c in range(NCH):
            def drain(k, _):
                pltpu.make_async_copy(emb_hbm.at[pl.ds(0, 1)],
                                      slab_v.at[pl.ds(k, 1)], gsem).wait()
                return _
            lax.fori_loop(0, CH, drain, None)

            def extract(g16, _, c=c):
                n0 = c * CH + g16 * L
                r8v = jnp.bitwise_and(idx_v[pl.ds(n0, L)], 7)
                sv = scale_v[pl.ds(n0, L)]
                for i in range(L):
                    k = g16 * L + i
                    sb = jnp.full((L,), sv[i], jnp.float32)
                    r8 = r8v[i]
                    for j in range(D // L):
                        sl = pl.ds(j * L, L)
                        stage_v[k, sl] = slab_v[k, r8, sl] * sb
                return _

            lax.fori_loop(0, CH // L, extract, None)
            pltpu.sync_copy(stage_v, out_hbm.at[pl.ds(base + c * CH, CH)])
            if c + 1 < NCH:
                fire(c + 1)

    return emb_kernel


_emb_kernel = _make_kernel()


@jax.jit
def kernel(x, mask, tag_id, emb_table, tag_table):
    x = x.astype(jnp.int32)
    maskf = mask.astype(jnp.float32)
    tag_id = tag_id.astype(jnp.int32)
    t, td = tag_table.shape
    tag_pad = jnp.zeros((T_PAD, T_PAD), jnp.float32).at[:t, :td].set(tag_table)
    emb_slabs = emb_table.reshape(V // SLAB, SLAB, D)
    return _emb_kernel(x, maskf, tag_id, tag_pad.reshape(-1), emb_slabs)
